# Initial kernel scaffold; baseline (speedup 1.0000x reference)
import jax
import jax.numpy as jnp
from jax.experimental import pallas as pl


def kernel(input_tensor, w1, b1, w2, b2, ind0, ind1, ind2, mix_ind):
    raise NotImplementedError("write your pallas kernel here")


# speedup vs baseline: 9.1705x; 9.1705x over previous
import functools

import jax
import jax.numpy as jnp
from jax import lax
from jax.experimental import pallas as pl
from jax.experimental.pallas import tpu as pltpu
from jax.experimental.pallas import tpu_sc as plsc

FI, HID, FO = 16, 32, 16
RR, CC = 64, 10000
S, L, M = 2, 8192, 8192
SM = S * M
W = RR * FI
NC, NS = 2, 16
NW = NC * NS
NE = 3 * SM
GCH = 16
RPT = SM // NW
NCH_A = RPT // GCH
JPT = L // NW
BMAX = 4096
TM = 256
FLUSH = 32

def _wid():
    return lax.axis_index("s") * NC + lax.axis_index("c")


def _gather_body(xt_hbm, idx_hbm, c0_hbm, c1_hbm, c2_hbm, idx_v, buf, gsem):
    wid = _wid()
    outs = (c0_hbm, c1_hbm, c2_hbm)
    for i in range(3):
        pltpu.sync_copy(idx_hbm.at[i * NW + wid], idx_v)
        out = outs[i]

        def chunk(ch, _, out=out):
            pltpu.async_copy(xt_hbm.at[idx_v.at[ch]], buf, gsem).wait()
            row0 = pl.multiple_of(wid * RPT + ch * GCH, GCH)
            pltpu.sync_copy(buf, out.at[pl.ds(row0, GCH)])
            return 0

        lax.fori_loop(0, NCH_A, chunk, 0)


def _scatter_max_body(
    d_hbm, srcp_hbm, starts_hbm, ot_hbm, starts_v, srcs_v, rows_v, stage_v, gsem
):
    wid = _wid()
    jlo = wid * JPT
    pltpu.sync_copy(starts_hbm.at[pl.ds(pl.multiple_of(jlo, 256), 272)], starts_v)
    e_lo = starts_v[pl.ds(0, 16)][0]
    al = pl.multiple_of((e_lo // 8) * 8, 8)
    pltpu.sync_copy(srcp_hbm.at[pl.ds(al, BMAX + 16)], srcs_v)

    def column(c, _):
        sv = starts_v[pl.ds(c, 16)]
        q0 = sv[0] - al
        n = sv[1] - sv[0]
        slot = lax.rem(c, FLUSH)

        def zero_k(k, _):
            stage_v[slot, pl.ds(k * 16, 16)] = jnp.zeros((16,), jnp.float32)
            return 0

        lax.fori_loop(0, W // 16, zero_k, 0)

        def chunk(ch, _):
            iv = srcs_v[pl.ds(q0 + ch * GCH, GCH)]
            pltpu.async_copy(d_hbm.at[iv], rows_v, gsem).wait()
            kk = jnp.minimum(n - ch * GCH, GCH)

            def edge(v, _):
                def maxk(k, _):
                    sl = pl.ds(k * 16, 16)
                    stage_v[slot, sl] = jnp.maximum(
                        stage_v[slot, sl], rows_v[v, sl]
                    )
                    return 0

                lax.fori_loop(0, W // 16, maxk, 0)
                return 0

            lax.fori_loop(0, kk, edge, 0)
            return 0

        lax.fori_loop(0, (n + GCH - 1) // GCH, chunk, 0)

        @pl.when(slot == FLUSH - 1)
        def _flush():
            pltpu.sync_copy(
                stage_v,
                ot_hbm.at[pl.ds(pl.multiple_of(jlo + c - (FLUSH - 1), FLUSH), FLUSH)],
            )

        return 0

    lax.fori_loop(0, JPT, column, 0)


@functools.cache
def _sc_kernels():
    mesh = plsc.VectorSubcoreMesh(
        core_axis_name="c", subcore_axis_name="s",
        num_cores=NC, num_subcores=NS,
    )
    gather = pl.kernel(
        _gather_body,
        out_type=(
            jax.ShapeDtypeStruct((SM, W), jnp.float32),
            jax.ShapeDtypeStruct((SM, W), jnp.float32),
            jax.ShapeDtypeStruct((SM, W), jnp.float32),
        ),
        mesh=mesh,
        scratch_types=[
            pltpu.VMEM((NCH_A, GCH), jnp.int32),
            pltpu.VMEM((GCH, W), jnp.float32),
            pltpu.SemaphoreType.DMA,
        ],
    )
    scatter_max = pl.kernel(
        _scatter_max_body,
        out_type=jax.ShapeDtypeStruct((L, W), jnp.float32),
        mesh=mesh,
        scratch_types=[
            pltpu.VMEM((272,), jnp.int32),
            pltpu.VMEM((BMAX + 16,), jnp.int32),
            pltpu.VMEM((GCH, W), jnp.float32),
            pltpu.VMEM((FLUSH, W), jnp.float32),
            pltpu.SemaphoreType.DMA,
        ],
    )
    return gather, scatter_max


def _mlp_body(c0_ref, c1_ref, c2_ref, w1e_ref, b1e_ref, w2e_ref, b2e_ref,
              d_ref):
    z = jnp.broadcast_to(b1e_ref[...], (TM * 8, 8 * HID))
    for i, c_ref in enumerate((c0_ref, c1_ref, c2_ref)):
        a_i = c_ref[...].reshape(TM * 8, 128)
        z = z + lax.dot_general(
            a_i, w1e_ref[...][i],
            ((( 1,), (0,)), ((), ())),
            preferred_element_type=jnp.float32,
            precision=lax.Precision.HIGHEST,
        )
    a = jnp.maximum(z, 0.0)
    d = lax.dot_general(
        a, w2e_ref[...],
        ((( 1,), (0,)), ((), ())),
        preferred_element_type=jnp.float32,
        precision=lax.Precision.HIGHEST,
    ) + b2e_ref[...]
    d_ref[...] = d.reshape(TM, W)


def _mlp(c0, c1, c2, w1e, b1e, w2e, b2e):
    cspec = pl.BlockSpec((TM, W), lambda t: (t, 0))
    return pl.pallas_call(
        _mlp_body,
        grid=(SM // TM,),
        in_specs=[
            cspec, cspec, cspec,
            pl.BlockSpec((3, 128, 8 * HID), lambda t: (0, 0, 0)),
            pl.BlockSpec((1, 8 * HID), lambda t: (0, 0)),
            pl.BlockSpec((8 * HID, 128), lambda t: (0, 0)),
            pl.BlockSpec((1, 128), lambda t: (0, 0)),
        ],
        out_specs=pl.BlockSpec((TM, W), lambda t: (t, 0)),
        out_shape=jax.ShapeDtypeStruct((SM, W), jnp.float32),
    )(c0, c1, c2, w1e, b1e, w2e, b2e)


def _expand_weights(w1, b1, w2, b2):
    e8 = jnp.eye(8, dtype=jnp.float32)
    w1i = w1.reshape(HID, 3, FI).transpose(1, 2, 0)
    w1e = jnp.einsum('pq,ifh->ipfqh', e8, w1i).reshape(3, 128, 8 * HID)
    b1e = jnp.tile(b1, 8).reshape(1, 8 * HID)
    w2e = jnp.einsum('pq,ho->phqo', e8, w2.T).reshape(8 * HID, 128)
    b2e = jnp.tile(b2, 8).reshape(1, 128)
    return w1e, b1e, w2e, b2e


def kernel(input_tensor, w1, b1, w2, b2, ind0, ind1, ind2, mix_ind):
    inds = jnp.stack([ind0, ind1, ind2], axis=1)

    k = jnp.arange(L, dtype=jnp.int32)
    code = k[None, None, :] * L + inds[..., 1]
    tm = jnp.full((S, 3, L), -1, dtype=jnp.int32)
    tm = tm.at[
        jnp.arange(S)[:, None, None], jnp.arange(3)[None, :, None], inds[..., 0]
    ].max(code)
    gcol = jnp.take_along_axis(tm, mix_ind, axis=2)
    gcol = jnp.where(gcol < 0, L, jnp.remainder(gcol, L))
    dst = jnp.take_along_axis(inds[..., 1], mix_ind, axis=2)

    xt = jnp.transpose(input_tensor[:, :, :L], (2, 1, 0)).reshape(L, W)
    xt = jnp.concatenate([xt, jnp.zeros((1, W), xt.dtype)], axis=0)

    idx_a = gcol.transpose(1, 0, 2).reshape(3 * NW, NCH_A, GCH)

    gather_kernel, scatter_max_kernel = _sc_kernels()
    c0, c1, c2 = gather_kernel(xt, idx_a)
    w1e, b1e, w2e, b2e = _expand_weights(w1, b1, w2, b2)
    d = _mlp(c0, c1, c2, w1e, b1e, w2e, b2e)

    dst_flat = dst.reshape(-1)
    src_flat = jnp.broadcast_to(
        (jnp.arange(S, dtype=jnp.int32) * M)[:, None, None]
        + jnp.arange(M, dtype=jnp.int32)[None, None, :],
        (S, 3, M),
    ).reshape(-1)
    order = jnp.argsort(dst_flat)
    src_sorted = src_flat[order]
    counts = jnp.zeros((L,), jnp.int32).at[dst_flat].add(1)
    starts = jnp.concatenate(
        [jnp.zeros((1,), jnp.int32), jnp.cumsum(counts, dtype=jnp.int32)]
    )
    srcp = jnp.concatenate(
        [src_sorted, jnp.zeros((BMAX + 24,), jnp.int32)]
    )
    starts_p = jnp.concatenate(
        [starts, jnp.full((272,), NE, jnp.int32)]
    )

    ot = scatter_max_kernel(d, srcp, starts_p)

    out = jnp.transpose(ot.reshape(L, RR, FO), (2, 1, 0))
    return jnp.concatenate(
        [out, jnp.zeros((FO, RR, CC - L), out.dtype)], axis=2
    )


# speedup vs baseline: 10.5094x; 1.1460x over previous
import functools

import jax
import jax.numpy as jnp
from jax import lax
from jax.experimental import pallas as pl
from jax.experimental.pallas import tpu as pltpu
from jax.experimental.pallas import tpu_sc as plsc

FI, HID, FO = 16, 32, 16
RR, CC = 64, 10000
S, L, M = 2, 8192, 8192
SM = S * M
W = RR * FI
NC, NS = 2, 16
NW = NC * NS
NE = 3 * SM
GCH = 16
RPT = SM // NW
NCH_A = RPT // GCH
JPT = L // NW
BMAX = 4096
TM = 256
FLUSH = 32

def _wid():
    return lax.axis_index("s") * NC + lax.axis_index("c")


NBUF_A = 4


def _gather_body(xt_hbm, idx_hbm, c0_hbm, c1_hbm, c2_hbm, idx_v, bufs,
                 g0, g1, g2, g3, w0, w1, w2, w3):
    gsems = (g0, g1, g2, g3)
    wsems = (w0, w1, w2, w3)
    wid = _wid()
    outs = (c0_hbm, c1_hbm, c2_hbm)

    def out_slice(out, ch):
        row0 = pl.multiple_of(wid * RPT + ch * GCH, GCH)
        return out.at[pl.ds(row0, GCH)]

    for i in range(3):
        pltpu.sync_copy(idx_hbm.at[i * NW + wid], idx_v)
        out = outs[i]
        for j in range(NBUF_A):
            pltpu.async_copy(xt_hbm.at[idx_v.at[j]], bufs.at[j], gsems[j])

        def kbody(k, _, out=out):
            for j in range(NBUF_A):
                ch = k * NBUF_A + j
                pltpu.make_async_copy(
                    xt_hbm.at[idx_v.at[ch]], bufs.at[j], gsems[j]
                ).wait()
                wd = pltpu.async_copy(bufs.at[j], out_slice(out, ch), wsems[j])

                @pl.when(k < NCH_A // NBUF_A - 1)
                def _next(j=j, ch=ch, wd=wd):
                    wd.wait()
                    pltpu.async_copy(
                        xt_hbm.at[idx_v.at[ch + NBUF_A]], bufs.at[j], gsems[j]
                    )

            return 0

        lax.fori_loop(0, NCH_A // NBUF_A, kbody, 0)
        for j in range(NBUF_A):
            pltpu.make_async_copy(
                bufs.at[j], out_slice(out, NCH_A - NBUF_A + j), wsems[j]
            ).wait()


def _scatter_max_body(
    d_hbm, srcp_hbm, starts_hbm, ot_hbm,
    starts_v, srcs_v, rowsA, rowsB, stage_v, semA, semB
):
    wid = _wid()
    jlo = wid * JPT
    pltpu.sync_copy(starts_hbm.at[pl.ds(pl.multiple_of(jlo, 256), 272)], starts_v)
    e_lo = starts_v[pl.ds(0, 16)][0]
    al = pl.multiple_of((e_lo // 8) * 8, 8)
    pltpu.sync_copy(srcp_hbm.at[pl.ds(al, BMAX + 16)], srcs_v)

    def col_meta(c):
        sv = starts_v[pl.ds(c, 16)]
        q0 = jnp.minimum(sv[0] - al, BMAX)
        return q0, sv[1] - sv[0]

    def issue(q0, buf, sem):
        iv = srcs_v[pl.ds(q0, GCH)]
        pltpu.async_copy(d_hbm.at[iv], buf, sem)

    def drain(buf, sem):
        pltpu.make_async_copy(
            d_hbm.at[jnp.zeros((GCH,), jnp.int32)], buf, sem
        ).wait()

    q0_0, n_0 = col_meta(0)
    issue(q0_0, rowsA, semA)

    def kbody(k, carry):
        q0, n = carry
        for j, (cur, csem, nxt, nsem) in enumerate(
            ((rowsA, semA, rowsB, semB), (rowsB, semB, rowsA, semA))
        ):
            c = 2 * k + j
            q0n, nn = col_meta(c + 1)
            issue(q0n, nxt, nsem)
            drain(cur, csem)
            slot = lax.rem(c, FLUSH)
            kk = jnp.minimum(n, GCH)

            def edge(v, _, cur=cur, slot=slot):
                for kq in range(W // 16):
                    sl = pl.ds(kq * 16, 16)
                    stage_v[slot, sl] = jnp.maximum(stage_v[slot, sl], cur[v, sl])
                return 0

            for kq in range(W // 16):
                stage_v[slot, pl.ds(kq * 16, 16)] = jnp.zeros((16,), jnp.float32)
            lax.fori_loop(0, kk, edge, 0)

            def extra_chunk(ch, _, cur=cur, csem=csem, q0=q0, n=n, slot=slot):
                iv = srcs_v[pl.ds(jnp.minimum(q0 + ch * GCH, BMAX), GCH)]
                pltpu.async_copy(d_hbm.at[iv], cur, csem).wait()
                kk2 = jnp.minimum(n - ch * GCH, GCH)

                def edge2(v, _):
                    for kq in range(W // 16):
                        sl = pl.ds(kq * 16, 16)
                        stage_v[slot, sl] = jnp.maximum(
                            stage_v[slot, sl], cur[v, sl]
                        )
                    return 0

                lax.fori_loop(0, kk2, edge2, 0)
                return 0

            lax.fori_loop(1, (n + GCH - 1) // GCH, extra_chunk, 0)

            @pl.when(slot == FLUSH - 1)
            def _flush(c=c):
                pltpu.sync_copy(
                    stage_v,
                    ot_hbm.at[
                        pl.ds(pl.multiple_of(jlo + c - (FLUSH - 1), FLUSH), FLUSH)
                    ],
                )

            q0, n = q0n, nn
        return q0, n

    lax.fori_loop(0, JPT // 2, kbody, (q0_0, n_0))
    drain(rowsA, semA)


@functools.cache
def _sc_kernels():
    mesh = plsc.VectorSubcoreMesh(
        core_axis_name="c", subcore_axis_name="s",
        num_cores=NC, num_subcores=NS,
    )
    gather = pl.kernel(
        _gather_body,
        out_type=(
            jax.ShapeDtypeStruct((SM, W), jnp.float32),
            jax.ShapeDtypeStruct((SM, W), jnp.float32),
            jax.ShapeDtypeStruct((SM, W), jnp.float32),
        ),
        mesh=mesh,
        scratch_types=[
            pltpu.VMEM((NCH_A, GCH), jnp.int32),
            pltpu.VMEM((NBUF_A, GCH, W), jnp.float32),
        ] + [pltpu.SemaphoreType.DMA] * (2 * NBUF_A),
    )
    scatter_max = pl.kernel(
        _scatter_max_body,
        out_type=jax.ShapeDtypeStruct((L, W), jnp.float32),
        mesh=mesh,
        scratch_types=[
            pltpu.VMEM((272,), jnp.int32),
            pltpu.VMEM((BMAX + 16,), jnp.int32),
            pltpu.VMEM((GCH, W), jnp.float32),
            pltpu.VMEM((GCH, W), jnp.float32),
            pltpu.VMEM((FLUSH, W), jnp.float32),
            pltpu.SemaphoreType.DMA,
            pltpu.SemaphoreType.DMA,
        ],
    )
    return gather, scatter_max


def _mlp_body(c0_ref, c1_ref, c2_ref, w1e_ref, b1e_ref, w2e_ref, b2e_ref,
              d_ref):
    z = jnp.broadcast_to(b1e_ref[...], (TM * 8, 8 * HID))
    for i, c_ref in enumerate((c0_ref, c1_ref, c2_ref)):
        a_i = c_ref[...].reshape(TM * 8, 128)
        z = z + lax.dot_general(
            a_i, w1e_ref[...][i],
            ((( 1,), (0,)), ((), ())),
            preferred_element_type=jnp.float32,
            precision=lax.Precision.HIGHEST,
        )
    a = jnp.maximum(z, 0.0)
    d = lax.dot_general(
        a, w2e_ref[...],
        ((( 1,), (0,)), ((), ())),
        preferred_element_type=jnp.float32,
        precision=lax.Precision.HIGHEST,
    ) + b2e_ref[...]
    d_ref[...] = d.reshape(TM, W)


def _mlp(c0, c1, c2, w1e, b1e, w2e, b2e):
    cspec = pl.BlockSpec((TM, W), lambda t: (t, 0))
    return pl.pallas_call(
        _mlp_body,
        grid=(SM // TM,),
        in_specs=[
            cspec, cspec, cspec,
            pl.BlockSpec((3, 128, 8 * HID), lambda t: (0, 0, 0)),
            pl.BlockSpec((1, 8 * HID), lambda t: (0, 0)),
            pl.BlockSpec((8 * HID, 128), lambda t: (0, 0)),
            pl.BlockSpec((1, 128), lambda t: (0, 0)),
        ],
        out_specs=pl.BlockSpec((TM, W), lambda t: (t, 0)),
        out_shape=jax.ShapeDtypeStruct((SM, W), jnp.float32),
    )(c0, c1, c2, w1e, b1e, w2e, b2e)


def _expand_weights(w1, b1, w2, b2):
    e8 = jnp.eye(8, dtype=jnp.float32)
    w1i = w1.reshape(HID, 3, FI).transpose(1, 2, 0)
    w1e = jnp.einsum('pq,ifh->ipfqh', e8, w1i).reshape(3, 128, 8 * HID)
    b1e = jnp.tile(b1, 8).reshape(1, 8 * HID)
    w2e = jnp.einsum('pq,ho->phqo', e8, w2.T).reshape(8 * HID, 128)
    b2e = jnp.tile(b2, 8).reshape(1, 128)
    return w1e, b1e, w2e, b2e


def kernel(input_tensor, w1, b1, w2, b2, ind0, ind1, ind2, mix_ind):
    inds = jnp.stack([ind0, ind1, ind2], axis=1)

    k = jnp.arange(L, dtype=jnp.int32)
    code = k[None, None, :] * L + inds[..., 1]
    tm = jnp.full((S, 3, L), -1, dtype=jnp.int32)
    tm = tm.at[
        jnp.arange(S)[:, None, None], jnp.arange(3)[None, :, None], inds[..., 0]
    ].max(code)
    gcol = jnp.take_along_axis(tm, mix_ind, axis=2)
    gcol = jnp.where(gcol < 0, L, jnp.remainder(gcol, L))
    dst = jnp.take_along_axis(inds[..., 1], mix_ind, axis=2)

    xt = jnp.transpose(input_tensor[:, :, :L], (2, 1, 0)).reshape(L, W)
    xt = jnp.concatenate([xt, jnp.zeros((1, W), xt.dtype)], axis=0)

    idx_a = gcol.transpose(1, 0, 2).reshape(3 * NW, NCH_A, GCH)

    gather_kernel, scatter_max_kernel = _sc_kernels()
    c0, c1, c2 = gather_kernel(xt, idx_a)
    w1e, b1e, w2e, b2e = _expand_weights(w1, b1, w2, b2)
    d = _mlp(c0, c1, c2, w1e, b1e, w2e, b2e)

    dst_flat = dst.reshape(-1)
    src_flat = jnp.broadcast_to(
        (jnp.arange(S, dtype=jnp.int32) * M)[:, None, None]
        + jnp.arange(M, dtype=jnp.int32)[None, None, :],
        (S, 3, M),
    ).reshape(-1)
    order = jnp.argsort(dst_flat)
    src_sorted = src_flat[order]
    counts = jnp.zeros((L,), jnp.int32).at[dst_flat].add(1)
    starts = jnp.concatenate(
        [jnp.zeros((1,), jnp.int32), jnp.cumsum(counts, dtype=jnp.int32)]
    )
    srcp = jnp.concatenate(
        [src_sorted, jnp.zeros((BMAX + 24,), jnp.int32)]
    )
    starts_p = jnp.concatenate(
        [starts, jnp.full((272,), NE, jnp.int32)]
    )

    ot = scatter_max_kernel(d, srcp, starts_p)

    out = jnp.transpose(ot.reshape(L, RR, FO), (2, 1, 0))
    return jnp.concatenate(
        [out, jnp.zeros((FO, RR, CC - L), out.dtype)], axis=2
    )


# speedup vs baseline: 11.9333x; 1.1355x over previous
import functools

import jax
import jax.numpy as jnp
from jax import lax
from jax.experimental import pallas as pl
from jax.experimental.pallas import tpu as pltpu
from jax.experimental.pallas import tpu_sc as plsc

FI, HID, FO = 16, 32, 16
RR, CC = 64, 10000
S, L, M = 2, 8192, 8192
SM = S * M
W = RR * FI
NC, NS = 2, 16
NW = NC * NS
NE = 3 * SM
GCH = 16
RPT = SM // NW
NCH_A = RPT // GCH
JPT = L // NW
BMAX = 4096
TM = 256
FLUSH = 32

def _wid():
    return lax.axis_index("s") * NC + lax.axis_index("c")


NBUF_A = 4


def _gather_body(xt_hbm, idx_hbm, c0_hbm, c1_hbm, c2_hbm, idx_v, bufs,
                 g0, g1, g2, g3, w0, w1, w2, w3):
    gsems = (g0, g1, g2, g3)
    wsems = (w0, w1, w2, w3)
    wid = _wid()
    outs = (c0_hbm, c1_hbm, c2_hbm)

    def out_slice(out, ch):
        row0 = pl.multiple_of(wid * RPT + ch * GCH, GCH)
        return out.at[pl.ds(row0, GCH)]

    for i in range(3):
        pltpu.sync_copy(idx_hbm.at[i * NW + wid], idx_v)
        out = outs[i]
        for j in range(NBUF_A):
            pltpu.async_copy(xt_hbm.at[idx_v.at[j]], bufs.at[j], gsems[j])

        def kbody(k, _, out=out):
            for j in range(NBUF_A):
                ch = k * NBUF_A + j
                pltpu.make_async_copy(
                    xt_hbm.at[idx_v.at[ch]], bufs.at[j], gsems[j]
                ).wait()
                wd = pltpu.async_copy(bufs.at[j], out_slice(out, ch), wsems[j])

                @pl.when(k < NCH_A // NBUF_A - 1)
                def _next(j=j, ch=ch, wd=wd):
                    wd.wait()
                    pltpu.async_copy(
                        xt_hbm.at[idx_v.at[ch + NBUF_A]], bufs.at[j], gsems[j]
                    )

            return 0

        lax.fori_loop(0, NCH_A // NBUF_A, kbody, 0)
        for j in range(NBUF_A):
            pltpu.make_async_copy(
                bufs.at[j], out_slice(out, NCH_A - NBUF_A + j), wsems[j]
            ).wait()


def _scatter_max_body(
    d_hbm, srcp_hbm, starts_hbm, ot_hbm,
    starts_v, srcs_v, rowsA, rowsB, stage_v, semA, semB
):
    wid = _wid()
    jlo = wid * JPT
    pltpu.sync_copy(starts_hbm.at[pl.ds(pl.multiple_of(jlo, 256), 272)], starts_v)
    e_lo = starts_v[pl.ds(0, 16)][0]
    al = pl.multiple_of((e_lo // 8) * 8, 8)
    pltpu.sync_copy(srcp_hbm.at[pl.ds(al, BMAX + 16)], srcs_v)

    def col_meta(c):
        sv = starts_v[pl.ds(c, 16)]
        q0 = jnp.minimum(sv[0] - al, BMAX)
        return q0, sv[1] - sv[0]

    def issue(q0, buf, sem):
        iv = srcs_v[pl.ds(q0, GCH)]
        pltpu.async_copy(d_hbm.at[iv], buf, sem)

    def drain(buf, sem):
        pltpu.make_async_copy(
            d_hbm.at[jnp.zeros((GCH,), jnp.int32)], buf, sem
        ).wait()

    q0_0, n_0 = col_meta(0)
    issue(q0_0, rowsA, semA)

    def kbody(k, carry):
        q0, n = carry
        for j, (cur, csem, nxt, nsem) in enumerate(
            ((rowsA, semA, rowsB, semB), (rowsB, semB, rowsA, semA))
        ):
            c = 2 * k + j
            q0n, nn = col_meta(c + 1)
            issue(q0n, nxt, nsem)
            drain(cur, csem)
            slot = lax.rem(c, FLUSH)
            kk = jnp.minimum(n, GCH)

            def edge(v, _, cur=cur, slot=slot):
                for sub in range(8):
                    for kq in range(4):
                        sl = pl.ds(kq * 32, 32)
                        stage_v[slot, sub, sl] = jnp.maximum(
                            stage_v[slot, sub, sl], cur[v, sub, sl]
                        )
                return 0

            for sub in range(8):
                for kq in range(4):
                    stage_v[slot, sub, pl.ds(kq * 32, 32)] = jnp.zeros(
                        (32,), jnp.bfloat16
                    )
            lax.fori_loop(0, kk, edge, 0)

            def extra_chunk(ch, _, cur=cur, csem=csem, q0=q0, n=n, slot=slot):
                iv = srcs_v[pl.ds(jnp.minimum(q0 + ch * GCH, BMAX), GCH)]
                pltpu.async_copy(d_hbm.at[iv], cur, csem).wait()
                kk2 = jnp.minimum(n - ch * GCH, GCH)

                def edge2(v, _):
                    for sub in range(8):
                        for kq in range(4):
                            sl = pl.ds(kq * 32, 32)
                            stage_v[slot, sub, sl] = jnp.maximum(
                                stage_v[slot, sub, sl], cur[v, sub, sl]
                            )
                    return 0

                lax.fori_loop(0, kk2, edge2, 0)
                return 0

            lax.fori_loop(1, (n + GCH - 1) // GCH, extra_chunk, 0)

            @pl.when(slot == FLUSH - 1)
            def _flush(c=c):
                pltpu.sync_copy(
                    stage_v,
                    ot_hbm.at[
                        pl.ds(pl.multiple_of(jlo + c - (FLUSH - 1), FLUSH), FLUSH)
                    ],
                )

            q0, n = q0n, nn
        return q0, n

    lax.fori_loop(0, JPT // 2, kbody, (q0_0, n_0))
    drain(rowsA, semA)


@functools.cache
def _sc_kernels():
    mesh = plsc.VectorSubcoreMesh(
        core_axis_name="c", subcore_axis_name="s",
        num_cores=NC, num_subcores=NS,
    )
    params = pltpu.CompilerParams(use_tc_tiling_on_sc=False)
    gather = pl.kernel(
        _gather_body,
        compiler_params=params,
        out_type=(
            jax.ShapeDtypeStruct((SM, 8, 128), jnp.bfloat16),
            jax.ShapeDtypeStruct((SM, 8, 128), jnp.bfloat16),
            jax.ShapeDtypeStruct((SM, 8, 128), jnp.bfloat16),
        ),
        mesh=mesh,
        scratch_types=[
            pltpu.VMEM((NCH_A, GCH), jnp.int32),
            pltpu.VMEM((NBUF_A, GCH, 8, 128), jnp.bfloat16),
        ] + [pltpu.SemaphoreType.DMA] * (2 * NBUF_A),
    )
    scatter_max = pl.kernel(
        _scatter_max_body,
        compiler_params=params,
        out_type=jax.ShapeDtypeStruct((L, 8, 128), jnp.bfloat16),
        mesh=mesh,
        scratch_types=[
            pltpu.VMEM((272,), jnp.int32),
            pltpu.VMEM((BMAX + 16,), jnp.int32),
            pltpu.VMEM((GCH, 8, 128), jnp.bfloat16),
            pltpu.VMEM((GCH, 8, 128), jnp.bfloat16),
            pltpu.VMEM((FLUSH, 8, 128), jnp.bfloat16),
            pltpu.SemaphoreType.DMA,
            pltpu.SemaphoreType.DMA,
        ],
    )
    return gather, scatter_max


def _mlp_body(c0_ref, c1_ref, c2_ref, w1e_ref, b1e_ref, w2e_ref, b2e_ref,
              d_ref):
    z = jnp.broadcast_to(b1e_ref[...], (TM * 8, 8 * HID))
    for i, c_ref in enumerate((c0_ref, c1_ref, c2_ref)):
        a_i = c_ref[...].reshape(TM * 8, 128)
        z = z + lax.dot_general(
            a_i, w1e_ref[...][i],
            ((( 1,), (0,)), ((), ())),
            preferred_element_type=jnp.float32,
        )
    a = jnp.maximum(z, 0.0).astype(jnp.bfloat16)
    d = lax.dot_general(
        a, w2e_ref[...],
        ((( 1,), (0,)), ((), ())),
        preferred_element_type=jnp.float32,
    ) + b2e_ref[...]
    d_ref[...] = d.astype(jnp.bfloat16).reshape(TM, 8, 128)


def _mlp(c0, c1, c2, w1e, b1e, w2e, b2e):
    cspec = pl.BlockSpec((TM, 8, 128), lambda t: (t, 0, 0))
    return pl.pallas_call(
        _mlp_body,
        grid=(SM // TM,),
        in_specs=[
            cspec, cspec, cspec,
            pl.BlockSpec((3, 128, 8 * HID), lambda t: (0, 0, 0)),
            pl.BlockSpec((1, 8 * HID), lambda t: (0, 0)),
            pl.BlockSpec((8 * HID, 128), lambda t: (0, 0)),
            pl.BlockSpec((1, 128), lambda t: (0, 0)),
        ],
        out_specs=pl.BlockSpec((TM, 8, 128), lambda t: (t, 0, 0)),
        out_shape=jax.ShapeDtypeStruct((SM, 8, 128), jnp.bfloat16),
    )(c0, c1, c2, w1e, b1e, w2e, b2e)


def _expand_weights(w1, b1, w2, b2):
    e8 = jnp.eye(8, dtype=jnp.float32)
    w1i = w1.reshape(HID, 3, FI).transpose(1, 2, 0)
    w1e = jnp.einsum('pq,ifh->ipfqh', e8, w1i).reshape(3, 128, 8 * HID)
    b1e = jnp.tile(b1, 8).reshape(1, 8 * HID)
    w2e = jnp.einsum('pq,ho->phqo', e8, w2.T).reshape(8 * HID, 128)
    b2e = jnp.tile(b2, 8).reshape(1, 128)
    return (w1e.astype(jnp.bfloat16), b1e,
            w2e.astype(jnp.bfloat16), b2e)


def kernel(input_tensor, w1, b1, w2, b2, ind0, ind1, ind2, mix_ind):
    inds = jnp.stack([ind0, ind1, ind2], axis=1)

    k = jnp.arange(L, dtype=jnp.int32)
    code = k[None, None, :] * L + inds[..., 1]
    tm = jnp.full((S, 3, L), -1, dtype=jnp.int32)
    tm = tm.at[
        jnp.arange(S)[:, None, None], jnp.arange(3)[None, :, None], inds[..., 0]
    ].max(code)
    gcol = jnp.take_along_axis(tm, mix_ind, axis=2)
    gcol = jnp.where(gcol < 0, L, jnp.remainder(gcol, L))
    dst = jnp.take_along_axis(inds[..., 1], mix_ind, axis=2)

    xt = jnp.transpose(
        input_tensor[:, :, :L].astype(jnp.bfloat16), (2, 1, 0)
    ).reshape(L, 8, 128)
    xt = jnp.concatenate([xt, jnp.zeros((1, 8, 128), xt.dtype)], axis=0)

    idx_a = gcol.transpose(1, 0, 2).reshape(3 * NW, NCH_A, GCH)

    gather_kernel, scatter_max_kernel = _sc_kernels()
    c0, c1, c2 = gather_kernel(xt, idx_a)
    w1e, b1e, w2e, b2e = _expand_weights(w1, b1, w2, b2)
    d = _mlp(c0, c1, c2, w1e, b1e, w2e, b2e)

    dst_flat = dst.reshape(-1)
    src_flat = jnp.broadcast_to(
        (jnp.arange(S, dtype=jnp.int32) * M)[:, None, None]
        + jnp.arange(M, dtype=jnp.int32)[None, None, :],
        (S, 3, M),
    ).reshape(-1)
    order = jnp.argsort(dst_flat)
    src_sorted = src_flat[order]
    counts = jnp.zeros((L,), jnp.int32).at[dst_flat].add(1)
    starts = jnp.concatenate(
        [jnp.zeros((1,), jnp.int32), jnp.cumsum(counts, dtype=jnp.int32)]
    )
    srcp = jnp.concatenate(
        [src_sorted, jnp.zeros((BMAX + 24,), jnp.int32)]
    )
    starts_p = jnp.concatenate(
        [starts, jnp.full((272,), NE, jnp.int32)]
    )

    ot = scatter_max_kernel(d, srcp, starts_p)

    out = jnp.transpose(ot.reshape(L, RR, FO), (2, 1, 0)).astype(jnp.float32)
    return jnp.concatenate(
        [out, jnp.zeros((FO, RR, CC - L), out.dtype)], axis=2
    )
